# baseline (speedup 1.0000x reference)
import functools

import jax
import jax.numpy as jnp
from jax import lax
from jax.experimental import pallas as pl
from jax.experimental.pallas import tpu as pltpu
from jax.experimental.pallas import tpu_sc as plsc

_B, _N, _NB = 16, 4096, 32
_RMIN2 = 0.1 * 0.1
_RMAX2 = 2.0 * 2.0

_NW = 32
_WPB = _NW // _B
_ROWS = _N // _WPB
_CH = 1024
_NCH = _ROWS // _CH


def _sc_body(pos_hbm, nbr_hbm, msk_hbm, out_hbm,
             pos_v, nb_v, mk_v, acc_v):
    c = lax.axis_index("c")
    s = lax.axis_index("s")
    wid = c * 16 + s
    b = wid // _WPB
    row0 = (wid % _WPB) * _ROWS

    pltpu.sync_copy(pos_hbm.at[b], pos_v)

    acc = jnp.zeros((16,), jnp.float32)
    for chunk in range(_NCH):
        r0 = row0 + chunk * _CH
        pltpu.sync_copy(nbr_hbm.at[b, pl.ds(r0 * _NB, _CH * _NB)], nb_v)
        pltpu.sync_copy(msk_hbm.at[b, pl.ds(r0 * _NB, _CH * _NB)], mk_v)

        def row_body(i, acc, _r0=r0):
            r3 = (_r0 + i) * 3
            cx = plsc.load_gather(pos_v, [jnp.full((16,), r3, jnp.int32)])
            cy = plsc.load_gather(pos_v, [jnp.full((16,), r3 + 1, jnp.int32)])
            cz = plsc.load_gather(pos_v, [jnp.full((16,), r3 + 2, jnp.int32)])
            for j in range(_NB // 16):
                idx = nb_v[pl.ds(i * _NB + j * 16, 16)]
                m = mk_v[pl.ds(i * _NB + j * 16, 16)]
                fx = idx * 3
                nx = plsc.load_gather(pos_v, [fx])
                ny = plsc.load_gather(pos_v, [fx + 1])
                nz = plsc.load_gather(pos_v, [fx + 2])
                dx = nx - cx
                dy = ny - cy
                dz = nz - cz
                sq = dx * dx + dy * dy + dz * dz
                valid = (m != 0) & (sq >= _RMIN2) & (sq <= _RMAX2)
                sq_safe = jnp.where(valid, sq, 1.0)
                acc = acc + jnp.where(valid, 1.0 / sq_safe, 0.0)
            return acc

        acc = lax.fori_loop(0, _CH, row_body, acc)

    acc_v[...] = acc
    pltpu.sync_copy(acc_v, out_hbm.at[wid])


def kernel(positions, cell, neighbors, offsets, mask):
    del cell, offsets
    pos = positions.reshape(_B, _N * 3)
    nbr = neighbors.reshape(_B, _N * _NB)
    msk = mask.reshape(_B, _N * _NB)

    mesh = plsc.VectorSubcoreMesh(core_axis_name="c", subcore_axis_name="s")
    run = functools.partial(
        pl.kernel,
        mesh=mesh,
        out_type=jax.ShapeDtypeStruct((_NW, 16), jnp.float32),
        compiler_params=pltpu.CompilerParams(needs_layout_passes=False),
        scratch_types=[
            pltpu.VMEM((_N * 3,), jnp.float32),
            pltpu.VMEM((_CH * _NB,), jnp.int32),
            pltpu.VMEM((_CH * _NB,), jnp.int32),
            pltpu.VMEM((16,), jnp.float32),
        ],
    )(_sc_body)
    partials = run(pos, nbr, msk)
    return partials.reshape(_B, _WPB, 16).sum(axis=(1, 2)) * 0.5


# speedup vs baseline: 1.5868x; 1.5868x over previous
import functools

import jax
import jax.numpy as jnp
from jax import lax
from jax.experimental import pallas as pl
from jax.experimental.pallas import tpu as pltpu
from jax.experimental.pallas import tpu_sc as plsc

_B, _N, _NB = 16, 4096, 32
_RMIN2 = 0.1 * 0.1
_RMAX2 = 2.0 * 2.0

_NW = 32
_WPB = _NW // _B
_ROWS = _N // _WPB
_PROWS = _B * _N * _NB // 128
_PPB = _PROWS // _B
_CHR = 256
_NCH = _PPB // _WPB // _CHR
_APC = _CHR * 128 // _NB


def _sc_body(pos_hbm, cmb_hbm, out_hbm, px_v, py_v, pz_v, cb_v, acc_v):
    c = lax.axis_index("c")
    s = lax.axis_index("s")
    wid = c * 16 + s
    b = wid // _WPB
    half = wid % _WPB

    pltpu.sync_copy(pos_hbm.at[0, pl.ds(b * 32, 32)], px_v)
    pltpu.sync_copy(pos_hbm.at[1, pl.ds(b * 32, 32)], py_v)
    pltpu.sync_copy(pos_hbm.at[2, pl.ds(b * 32, 32)], pz_v)

    acc = jnp.zeros((16,), jnp.float32)
    for chunk in range(_NCH):
        base_row = b * _PPB + half * (_PPB // _WPB) + chunk * _CHR
        atom0 = half * _ROWS + chunk * _APC
        pltpu.sync_copy(cmb_hbm.at[pl.ds(base_row, _CHR)], cb_v)

        def row_body(rr, acc, _atom0=atom0):
            acc_in = acc
            for l in range(8):
                if l % 2 == 0:
                    a = _atom0 + rr * 4 + l // 2
                    qc = jnp.full((16,), a >> 7, jnp.int32)
                    rc = jnp.full((16,), a & 127, jnp.int32)
                    cx = plsc.load_gather(px_v, [qc, rc])
                    cy = plsc.load_gather(py_v, [qc, rc])
                    cz = plsc.load_gather(pz_v, [qc, rc])
                v = cb_v[rr, pl.ds(l * 16, 16)]
                idx = v & 4095
                q = idx >> 7
                r = idx & 127
                nx = plsc.load_gather(px_v, [q, r])
                ny = plsc.load_gather(py_v, [q, r])
                nz = plsc.load_gather(pz_v, [q, r])
                dx = nx - cx
                dy = ny - cy
                dz = nz - cz
                sq = dx * dx + dy * dy + dz * dz
                valid = (v >= 32768) & (sq >= _RMIN2) & (sq <= _RMAX2)
                acc_in = acc_in + jnp.where(valid, 1.0 / sq, 0.0)
            return acc_in

        acc = lax.fori_loop(0, _CHR, row_body, acc)

    acc_v[...] = acc
    pltpu.sync_copy(acc_v, out_hbm.at[wid])


def kernel(positions, cell, neighbors, offsets, mask):
    del cell, offsets
    cmb = (neighbors | (mask << 15)).reshape(_PROWS, 128)
    pos_t = positions.transpose((2, 0, 1)).reshape(3, _B * _N // 128, 128)

    mesh = plsc.VectorSubcoreMesh(core_axis_name="c", subcore_axis_name="s")
    run = functools.partial(
        pl.kernel,
        mesh=mesh,
        out_type=jax.ShapeDtypeStruct((_NW, 16), jnp.float32),
        compiler_params=pltpu.CompilerParams(needs_layout_passes=False),
        scratch_types=[
            pltpu.VMEM((32, 128), jnp.float32),
            pltpu.VMEM((32, 128), jnp.float32),
            pltpu.VMEM((32, 128), jnp.float32),
            pltpu.VMEM((_CHR, 128), jnp.int32),
            pltpu.VMEM((16,), jnp.float32),
        ],
    )(_sc_body)
    partials = run(pos_t, cmb)
    return partials.reshape(_B, _WPB, 16).sum(axis=(1, 2)) * 0.5
